# baseline (speedup 1.0000x reference)
import functools

import jax
import jax.numpy as jnp
from jax import lax
from jax.experimental import pallas as pl
from jax.experimental.pallas import tpu as pltpu
from jax.experimental.pallas import tpu_sc as plsc

_BATCH = 16384
_D = 32

_info = plsc.get_sparse_core_info()
_NC, _NS = _info.num_cores, _info.num_subcores
_NW = _NC * _NS
_BPW = _BATCH // _NW

_mesh = plsc.VectorSubcoreMesh(core_axis_name="c", subcore_axis_name="s")


@functools.partial(
    pl.kernel,
    mesh=_mesh,
    compiler_params=pltpu.CompilerParams(use_tc_tiling_on_sc=False),
    out_type=[
        jax.ShapeDtypeStruct((_BATCH, _D), jnp.float32),
        jax.ShapeDtypeStruct((_BATCH, _D), jnp.float32),
        jax.ShapeDtypeStruct((_BATCH, _D), jnp.float32),
    ],
    scratch_types=[
        pltpu.VMEM((_BPW,), jnp.int32),
        pltpu.VMEM((_BPW,), jnp.int32),
        pltpu.VMEM((_BPW,), jnp.int32),
        pltpu.VMEM((_BPW, _D), jnp.float32),
        pltpu.VMEM((_BPW, _D), jnp.float32),
        pltpu.VMEM((_BPW, _D), jnp.float32),
        pltpu.SemaphoreType.DMA,
        pltpu.SemaphoreType.DMA,
        pltpu.SemaphoreType.DMA,
    ],
)
def _gather3(idx_u, idx_i, idx_t, t_u, t_i, t_t,
             out_u, out_i, out_t,
             iv_u, iv_i, iv_t, rv_u, rv_i, rv_t,
             sem_u, sem_i, sem_t):
    wid = lax.axis_index("s") * _NC + lax.axis_index("c")
    base = wid * _BPW
    pltpu.sync_copy(idx_u.at[pl.ds(base, _BPW)], iv_u)
    pltpu.sync_copy(idx_i.at[pl.ds(base, _BPW)], iv_i)
    pltpu.sync_copy(idx_t.at[pl.ds(base, _BPW)], iv_t)
    cu = pltpu.async_copy(t_u.at[iv_u], rv_u, sem_u)
    ci = pltpu.async_copy(t_i.at[iv_i], rv_i, sem_i)
    ct = pltpu.async_copy(t_t.at[iv_t], rv_t, sem_t)
    cu.wait()
    pltpu.sync_copy(rv_u, out_u.at[pl.ds(base, _BPW)])
    ci.wait()
    pltpu.sync_copy(rv_i, out_i.at[pl.ds(base, _BPW)])
    ct.wait()
    pltpu.sync_copy(rv_t, out_t.at[pl.ds(base, _BPW)])


def kernel(idx_user, idx_item, idx_tag, T_user, T_item, T_tag):
    out = _gather3(idx_user, idx_item, idx_tag, T_user, T_item, T_tag)
    return (out[0], out[1], out[2])


# speedup vs baseline: 2.4027x; 2.4027x over previous
import functools

import jax
import jax.numpy as jnp
from jax import lax
from jax.experimental import pallas as pl
from jax.experimental.pallas import tpu as pltpu
from jax.experimental.pallas import tpu_sc as plsc

_BATCH = 16384
_D = 32

_info = plsc.get_sparse_core_info()
_NC, _NS = _info.num_cores, _info.num_subcores
_NW = _NC * _NS
_BPW = _BATCH // _NW

_mesh = plsc.VectorSubcoreMesh(core_axis_name="c", subcore_axis_name="s")


@functools.partial(
    pl.kernel,
    mesh=_mesh,
    compiler_params=pltpu.CompilerParams(use_tc_tiling_on_sc=True,
                                         needs_layout_passes=False),
    out_type=[
        jax.ShapeDtypeStruct((_BATCH, _D), jnp.float32),
        jax.ShapeDtypeStruct((_BATCH, _D), jnp.float32),
        jax.ShapeDtypeStruct((_BATCH, _D), jnp.float32),
    ],
    scratch_types=[
        pltpu.VMEM((_BPW,), jnp.int32),
        pltpu.VMEM((_BPW, _D), jnp.float32),
        pltpu.SemaphoreType.DMA,
        pltpu.SemaphoreType.DMA,
    ],
)
def _gather3(idx_u, idx_i, idx_t, t_u, t_i, t_t,
             out_u, out_i, out_t,
             iv, rb, sem, sem2):
    wid = lax.axis_index("s") * _NC + lax.axis_index("c")
    base = wid * _BPW

    for idx, tbl, out in ((idx_u, t_u, out_u),
                          (idx_i, t_i, out_i),
                          (idx_t, t_t, out_t)):
        pltpu.sync_copy(idx.at[pl.ds(base, _BPW)], iv)

        def issue_body(g, _, tbl=tbl):
            v16 = iv[pl.ds(g * 16, 16)]
            for e in range(16):
                i = v16[e]
                t = lax.shift_right_logical(i, 3)
                s = jnp.bitwise_and(i, 7)
                pltpu.async_copy(tbl.at[t, s], rb.at[g * 16 + e], sem)
            return _

        lax.fori_loop(0, _BPW // 16, issue_body, None)

        def drain_body(g, _, tbl=tbl):
            for e in range(16):
                pltpu.make_async_copy(tbl.at[0, 0], rb.at[0], sem).wait()
            return _

        lax.fori_loop(0, _BPW // 16, drain_body, None)
        pltpu.async_copy(rb, out.at[pl.ds(base, _BPW)], sem2).wait()


def kernel(idx_user, idx_item, idx_tag, T_user, T_item, T_tag):
    out = _gather3(
        idx_user, idx_item, idx_tag,
        T_user.reshape(-1, 8, _D),
        T_item.reshape(-1, 8, _D),
        T_tag.reshape(-1, 8, _D),
    )
    return (out[0], out[1], out[2])
